# Initial kernel scaffold; baseline (speedup 1.0000x reference)
import jax
import jax.numpy as jnp
from jax.experimental import pallas as pl


def kernel(superPoint_feat, rawPoint_feat, point_assignment):
    raise NotImplementedError("write your pallas kernel here")


# speedup vs baseline: 1.5782x; 1.5782x over previous
import jax
import jax.numpy as jnp
from jax import lax
from jax.experimental import pallas as pl
from jax.experimental.pallas import tpu as pltpu
from jax.experimental.pallas import tpu_sc as plsc

_N = 320000
_T = 10000
_D = 128
_W = 144
_BLK = 128
_NB = _N // _BLK
_NW = 32
_TP = 10112
_ROWS_PER_TILE = _TP // 16

_PRE_B = 2000


def _prepass_body(x_ref, o_ref):
    x = x_ref[...]
    ss = jnp.sum(x * x, axis=1, keepdims=True)
    scale = 1.0 / jnp.maximum(jnp.sqrt(ss), 1e-12)
    y = x * scale
    col16 = lax.broadcasted_iota(jnp.int32, (_PRE_B, 16), 1)
    flag = jnp.where(col16 == 0, 1.0, 0.0).astype(jnp.float32)
    o_ref[...] = jnp.concatenate([y, flag], axis=1)


def _normalize_tc(raw):
    return pl.pallas_call(
        _prepass_body,
        grid=(_N // _PRE_B,),
        in_specs=[pl.BlockSpec((_PRE_B, _D), lambda i: (i, 0))],
        out_specs=pl.BlockSpec((_PRE_B, _W), lambda i: (i, 0)),
        out_shape=jax.ShapeDtypeStruct((_N, _W), jnp.float32),
    )(raw)


def _sc_body(y_hbm, idx_hbm, z_hbm, out_hbm, idx_v, rows_v, table):
    c = lax.axis_index("c")
    s = lax.axis_index("s")
    w = c * 16 + s

    pltpu.sync_copy(
        z_hbm.at[pl.ds(s * _ROWS_PER_TILE, _ROWS_PER_TILE)],
        table.at[pl.ds(s * _ROWS_PER_TILE, _ROWS_PER_TILE)],
    )
    plsc.subcore_barrier()

    nblk = 78 + jnp.where(w < 4, 1, 0)
    start = 78 * w + jnp.minimum(w, 4)

    def body(j, carry):
        b = start + j
        pltpu.sync_copy(idx_hbm.at[pl.ds(b, 1)], idx_v)
        pltpu.sync_copy(y_hbm.at[pl.ds(b * _BLK, _BLK)], rows_v)
        pltpu.sync_copy(rows_v, table.at[idx_v.at[0]], add=True)
        return carry

    lax.fori_loop(0, nblk, body, 0)
    plsc.subcore_barrier()

    pltpu.sync_copy(
        table.at[pl.ds(s * _ROWS_PER_TILE, _ROWS_PER_TILE)],
        out_hbm.at[c, pl.ds(s * _ROWS_PER_TILE, _ROWS_PER_TILE)],
    )


_sc_scatter = pl.kernel(
    _sc_body,
    out_type=jax.ShapeDtypeStruct((2, _TP, _W), jnp.float32),
    mesh=plsc.VectorSubcoreMesh(core_axis_name="c", subcore_axis_name="s"),
    compiler_params=pltpu.CompilerParams(use_tc_tiling_on_sc=False),
    scratch_types=[
        pltpu.VMEM((1, _BLK), jnp.int32),
        pltpu.VMEM((_BLK, _W), jnp.float32),
        pltpu.VMEM_SHARED((_TP, _W), jnp.float32),
    ],
)


def _epilogue_body(sp_ref, t_ref, o_ref):
    sp = sp_ref[...]
    t = t_ref[0] + t_ref[1]
    seg_sum = t[:_T, :_D]
    counts = t[:_T, _D:_D + 1]

    ss = jnp.sum(sp * sp, axis=1, keepdims=True)
    spn = sp / jnp.maximum(jnp.sqrt(ss), 1e-12)

    cc = jnp.maximum(counts, 1.0)
    mean = seg_sum / cc

    dot = jnp.sum(spn * mean, axis=1)
    na = jnp.maximum(jnp.sqrt(jnp.sum(spn * spn, axis=1)), 1e-8)
    nb = jnp.maximum(jnp.sqrt(jnp.sum(mean * mean, axis=1)), 1e-8)
    cos = dot / (na * nb)
    weights = counts[:, 0] / float(_N)
    o_ref[...] = jnp.sum((1.0 - cos) * weights).reshape(1, 1)


def _epilogue_tc(sp, tables):
    return pl.pallas_call(
        _epilogue_body,
        out_shape=jax.ShapeDtypeStruct((1, 1), jnp.float32),
    )(sp, tables)


def kernel(superPoint_feat, rawPoint_feat, point_assignment):
    y = _normalize_tc(rawPoint_feat)
    idx = point_assignment.reshape(_NB, _BLK)
    zeros = jnp.zeros((_TP, _W), jnp.float32)
    tables = _sc_scatter(y, idx, zeros)
    loss = _epilogue_tc(superPoint_feat, tables)
    return loss[0, 0]


# speedup vs baseline: 4.7997x; 3.0412x over previous
import jax
import jax.numpy as jnp
from jax import lax
from jax.experimental import pallas as pl
from jax.experimental.pallas import tpu as pltpu
from jax.experimental.pallas import tpu_sc as plsc

_N = 320000
_T = 10000
_D = 128
_W = 144
_BLK = 128
_NB = _N // _BLK
_TP = 10112
_RPT = _TP // 16

_MAGIC = 0x5F3759DF


def _normalize_block(stage):
    def group16(g, _):
        for r in range(16):
            row = g * 16 + r
            vs = [stage[row, pl.ds(k * 16, 16)] for k in range(8)]
            acc = vs[0] * vs[0]
            for k in range(1, 8):
                acc = acc + vs[k] * vs[k]
            cs = plsc.cumsum(acc)
            x = cs[jnp.full((16,), 15, jnp.int32)]
            i = plsc.bitcast(x, jnp.int32)
            i = _MAGIC - lax.shift_right_logical(i, 1)
            y = plsc.bitcast(i, jnp.float32)
            for _ in range(3):
                y = y * (1.5 - 0.5 * x * y * y)
            for k in range(8):
                stage[row, pl.ds(k * 16, 16)] = vs[k] * y
        return 0

    lax.fori_loop(0, 8, group16, 0)


def _sc_body(raw_hbm, idx_hbm, out_hbm,
             idx_a, idx_b, stage_a, stage_b, table,
             sem_ia, sem_ib, sem_ja, sem_jb, sem_oa, sem_ob):
    c = lax.axis_index("c")
    s = lax.axis_index("s")
    w = c * 16 + s
    lane = lax.iota(jnp.int32, 16)

    zero16 = jnp.zeros((16,), jnp.float32)

    def zrow(r, _):
        for k in range(9):
            stage_a[r, pl.ds(k * 16, 16)] = zero16
            stage_b[r, pl.ds(k * 16, 16)] = zero16
        return 0

    lax.fori_loop(0, _BLK, zrow, 0)
    base_t = s * _RPT
    for t in range(4):
        pltpu.sync_copy(stage_a.at[pl.ds(0, 128)],
                        table.at[pl.ds(base_t + t * 128, 128)])
    pltpu.sync_copy(stage_a.at[pl.ds(0, 120)],
                    table.at[pl.ds(base_t + 512, 120)])

    flag16 = jnp.where(lane == 0, 1.0, 0.0).astype(jnp.float32)

    def frow(r, _):
        stage_a[r, pl.ds(_D, 16)] = flag16
        stage_b[r, pl.ds(_D, 16)] = flag16
        return 0

    lax.fori_loop(0, _BLK, frow, 0)
    plsc.subcore_barrier()

    start_blk = 78 * w + jnp.minimum(w, 4)

    def pair(p, _):
        b0 = start_blk + 2 * p
        in_a = pltpu.async_copy(raw_hbm.at[pl.ds(b0 * _BLK, _BLK)],
                                stage_a.at[:, pl.ds(0, _D)], sem_ia)
        ji_a = pltpu.async_copy(idx_hbm.at[pl.ds(b0, 1)], idx_a, sem_ja)
        in_b = pltpu.async_copy(raw_hbm.at[pl.ds((b0 + 1) * _BLK, _BLK)],
                                stage_b.at[:, pl.ds(0, _D)], sem_ib)
        ji_b = pltpu.async_copy(idx_hbm.at[pl.ds(b0 + 1, 1)], idx_b, sem_jb)
        in_a.wait()
        ji_a.wait()
        _normalize_block(stage_a)
        out_a = pltpu.async_copy(stage_a, table.at[idx_a.at[0]], sem_oa, add=True)
        in_b.wait()
        ji_b.wait()
        _normalize_block(stage_b)
        out_b = pltpu.async_copy(stage_b, table.at[idx_b.at[0]], sem_ob, add=True)
        out_a.wait()
        out_b.wait()
        return 0

    lax.fori_loop(0, 39, pair, 0)

    @pl.when(w < 4)
    def _tail():
        b = start_blk + 78
        pltpu.sync_copy(raw_hbm.at[pl.ds(b * _BLK, _BLK)],
                        stage_a.at[:, pl.ds(0, _D)])
        pltpu.sync_copy(idx_hbm.at[pl.ds(b, 1)], idx_a)
        _normalize_block(stage_a)
        pltpu.sync_copy(stage_a, table.at[idx_a.at[0]], add=True)

    plsc.subcore_barrier()
    pltpu.sync_copy(table.at[pl.ds(base_t, _RPT)],
                    out_hbm.at[c, pl.ds(base_t, _RPT)])


_sc_scatter = pl.kernel(
    _sc_body,
    out_type=jax.ShapeDtypeStruct((2, _TP, _W), jnp.float32),
    mesh=plsc.VectorSubcoreMesh(core_axis_name="c", subcore_axis_name="s"),
    compiler_params=pltpu.CompilerParams(use_tc_tiling_on_sc=False,
                                         needs_layout_passes=False),
    scratch_types=[
        pltpu.VMEM((1, _BLK), jnp.int32),
        pltpu.VMEM((1, _BLK), jnp.int32),
        pltpu.VMEM((_BLK, _W), jnp.float32),
        pltpu.VMEM((_BLK, _W), jnp.float32),
        pltpu.VMEM_SHARED((_TP, _W), jnp.float32),
        pltpu.SemaphoreType.DMA,
        pltpu.SemaphoreType.DMA,
        pltpu.SemaphoreType.DMA,
        pltpu.SemaphoreType.DMA,
        pltpu.SemaphoreType.DMA,
        pltpu.SemaphoreType.DMA,
    ],
)


def _epilogue_body(sp_ref, t_ref, o_ref):
    sp = sp_ref[...]
    t = t_ref[0] + t_ref[1]
    seg_sum = t[:_T, :_D]
    counts = t[:_T, _D:_D + 1]

    ss = jnp.sum(sp * sp, axis=1, keepdims=True)
    spn = sp / jnp.maximum(jnp.sqrt(ss), 1e-12)

    cc = jnp.maximum(counts, 1.0)
    mean = seg_sum / cc

    dot = jnp.sum(spn * mean, axis=1)
    na = jnp.maximum(jnp.sqrt(jnp.sum(spn * spn, axis=1)), 1e-8)
    nb = jnp.maximum(jnp.sqrt(jnp.sum(mean * mean, axis=1)), 1e-8)
    cos = dot / (na * nb)
    weights = counts[:, 0] / float(_N)
    o_ref[...] = jnp.sum((1.0 - cos) * weights).reshape(1, 1)


def _epilogue_tc(sp, tables):
    return pl.pallas_call(
        _epilogue_body,
        out_shape=jax.ShapeDtypeStruct((1, 1), jnp.float32),
    )(sp, tables)


def kernel(superPoint_feat, rawPoint_feat, point_assignment):
    idx = point_assignment.reshape(_NB, _BLK)
    tables = _sc_scatter(rawPoint_feat, idx)
    loss = _epilogue_tc(superPoint_feat, tables)
    return loss[0, 0]
